# Initial kernel scaffold; baseline (speedup 1.0000x reference)
import jax
import jax.numpy as jnp
from jax.experimental import pallas as pl


def kernel(x, edge_index, W1, a_src1, a_dst1, b1, ln1_g, ln1_b, W2, a_src2, a_dst2, b2, ln2_g, ln2_b, W3, a_src3, a_dst3, b3, lno_g, lno_b):
    raise NotImplementedError("write your pallas kernel here")


# speedup vs baseline: 18.8960x; 18.8960x over previous
import functools

import jax
import jax.numpy as jnp
from jax import lax
from jax.experimental import pallas as pl
from jax.experimental.pallas import tpu as pltpu
from jax.experimental.pallas import tpu_sc as plsc

N = 10000
E = 160000
F = 256
H = 4
C1 = 128
C2 = 40
C2P = 48
C3 = 40
C3P = 48
G1 = H * C1
G2P = H * C2P
NC = 2
NS = 16
CHUNK = 80
NCH = (E // NS) // CHUNK
NPAD = 10240
STRIPE = NPAD // NS
LAST_STRIPE = N - (NS - 1) * STRIPE

_HIGH = jax.lax.Precision.HIGHEST


def _dot(a, b):
    return jnp.dot(a, b, preferred_element_type=jnp.float32, precision=_HIGH)


def _mm1_body(x_ref, w_ref, a_ref, xp_ref, s_ref):
    xp = _dot(x_ref[:, :], w_ref[:, :])
    xp_ref[:, :] = xp
    s_ref[:, :] = _dot(xp, a_ref[:, :])


def _post_mm_body(nblocks, csplit, nreal, acc_ref, den_ref, b_ref, g_ref,
                  bl_ref, msk_ref, w_ref, a_ref, xp_ref, s_ref):
    den = den_ref[:, :]
    ys = []
    for hc in range(nblocks):
        h = hc // csplit
        d = den[:, h:h + 1] + 1e-16
        ys.append(acc_ref[hc, :, :] / d)
    y = jnp.concatenate(ys, axis=-1) + b_ref[:, :]
    inv = 1.0 / float(nreal)
    mu = jnp.sum(y, axis=-1, keepdims=True) * inv
    dlt = y - mu
    var = jnp.sum(msk_ref[:, :] * dlt * dlt, axis=-1, keepdims=True) * inv
    z = dlt * jax.lax.rsqrt(var + 1e-5) * g_ref[:, :] + bl_ref[:, :]
    act = jnp.where(z > 0, z, jnp.exp(z) - 1.0)
    xp = _dot(act, w_ref[:, :])
    xp_ref[:, :] = xp
    s_ref[:, :] = _dot(xp, a_ref[:, :])


def _post3_body(acc_ref, den_ref, b_ref, g_ref, bl_ref, msk_ref, out_ref):
    d = den_ref[:, :] + 1e-16
    y = acc_ref[0, :, :] / d + b_ref[:, :]
    inv = 1.0 / float(C3)
    mu = jnp.sum(y, axis=-1, keepdims=True) * inv
    dlt = y - mu
    var = jnp.sum(msk_ref[:, :] * dlt * dlt, axis=-1, keepdims=True) * inv
    z = dlt * jax.lax.rsqrt(var + 1e-5) * g_ref[:, :] + bl_ref[:, :]
    out_ref[:, :] = z[:, :C3]


_BN = 2000
_GRID = N // _BN


def _row_spec(cols):
    return pl.BlockSpec((_BN, cols), lambda i: (i, 0))


def _full_spec(r, c):
    return pl.BlockSpec((r, c), lambda i: (0, 0))


def _mm1(x, w, a):
    return pl.pallas_call(
        _mm1_body,
        grid=(_GRID,),
        in_specs=[_row_spec(F), _full_spec(F, G1), _full_spec(G1, 2 * H)],
        out_specs=[_row_spec(G1), _row_spec(2 * H)],
        out_shape=[jax.ShapeDtypeStruct((N, G1), jnp.float32),
                   jax.ShapeDtypeStruct((N, 2 * H), jnp.float32)],
    )(x, w, a)


def _post_mm(nblocks, csplit, nreal, cin, gout, nsl, acc, den, b, g, bl,
             msk, w, a):
    body = functools.partial(_post_mm_body, nblocks, csplit, nreal)
    gin = nblocks * cin
    nh = nblocks // csplit
    return pl.pallas_call(
        body,
        grid=(_GRID,),
        in_specs=[
            pl.BlockSpec((nblocks, _BN, cin), lambda i: (0, i, 0)),
            pl.BlockSpec((_BN, nh), lambda i: (i, 0)),
            _full_spec(1, gin), _full_spec(1, gin), _full_spec(1, gin),
            _full_spec(1, gin), _full_spec(gin, gout), _full_spec(gout, nsl),
        ],
        out_specs=[_row_spec(gout), _row_spec(nsl)],
        out_shape=[jax.ShapeDtypeStruct((N, gout), jnp.float32),
                   jax.ShapeDtypeStruct((N, nsl), jnp.float32)],
    )(acc, den, b, g, bl, msk, w, a)


def _post3(acc, den, b, g, bl, msk):
    return pl.pallas_call(
        _post3_body,
        grid=(_GRID,),
        in_specs=[
            pl.BlockSpec((1, _BN, C3P), lambda i: (0, i, 0)),
            pl.BlockSpec((_BN, 1), lambda i: (i, 0)),
            _full_spec(1, C3P), _full_spec(1, C3P), _full_spec(1, C3P),
            _full_spec(1, C3P),
        ],
        out_specs=_row_spec(C3),
        out_shape=jax.ShapeDtypeStruct((N, C3), jnp.float32),
    )(acc, den, b, g, bl, msk)


def _edge_body(cfg, src_hbm, dst_hbm, ssrc_hbm, sdst_hbm, xp_hbm,
               zrows_hbm, zden_hbm, *rest):
    hpc, heads_total, c, csplit, with_alpha = cfg
    if with_alpha:
        acc_out, den_out, alpha_out, srcb, dstb, exb, gidx, rows, sem, \
            acc_sh, den_sh = rest
    else:
        acc_out, den_out, srcb, dstb, exb, gidx, rows, sem, acc_sh, \
            den_sh = rest

    cid = lax.axis_index("c")
    sid = lax.axis_index("s")

    def tile_work():
        pltpu.sync_copy(src_hbm.at[sid], srcb)
        pltpu.sync_copy(dst_hbm.at[sid], dstb)

        def phase1(idxb, tab_hbm, finish):
            def inner(tab_v):
                pltpu.sync_copy(tab_hbm, tab_v)

                def jbody(j, carry):
                    for v in range(CHUNK // 16):
                        iv = idxb[j, pl.ds(v * 16, 16)]
                        for p in range(hpc):
                            h = cid * hpc + p
                            if heads_total > 1:
                                hv = jnp.full((16,), h, jnp.int32)
                                g = plsc.load_gather(tab_v, [hv, iv])
                            else:
                                g = plsc.load_gather(tab_v, [iv])
                            sl = (p, j, pl.ds(v * 16, 16))
                            if finish:
                                e = exb[sl] + g
                                e = jnp.where(e >= 0,
                                              e, e * jnp.float32(0.2))
                                exb[sl] = jnp.exp(e)
                            else:
                                exb[sl] = g
                    return carry

                lax.fori_loop(0, NCH, jbody, 0)

            if heads_total > 1:
                pl.run_scoped(
                    inner, pltpu.VMEM((heads_total, N), jnp.float32))
            else:
                pl.run_scoped(inner, pltpu.VMEM((N,), jnp.float32))

        phase1(srcb, ssrc_hbm, False)
        phase1(dstb, sdst_hbm, True)

        row0 = sid * STRIPE
        for p in range(hpc):
            h = cid * hpc + p
            for half in range(csplit):
                pltpu.sync_copy(zrows_hbm, acc_sh.at[pl.ds(row0, STRIPE)])
                if half == 0:
                    pltpu.sync_copy(zden_hbm,
                                    den_sh.at[pl.ds(row0, STRIPE)])
                plsc.subcore_barrier()

                def jbody2(j, carry):
                    for v in range(CHUNK // 16):
                        sv = srcb[j, pl.ds(v * 16, 16)]
                        gidx[pl.ds(v * 16, 16)] = (
                            (sv * heads_total + h) * csplit + half)

                    pltpu.async_copy(xp_hbm.at[gidx], rows, sem).wait()

                    def rbody(v, rc):
                        ex16 = exb[p, j, pl.ds(v * 16, 16)]
                        for r16 in range(16):
                            r = v * 16 + r16
                            s = ex16[r16]
                            for k in range(c // 16):
                                rows[r, pl.ds(k * 16, 16)] = (
                                    rows[r, pl.ds(k * 16, 16)] * s)
                        return rc

                    lax.fori_loop(0, CHUNK // 16, rbody, 0)
                    pltpu.sync_copy(rows, acc_sh.at[dstb.at[j]], add=True)
                    if half == 0:
                        pltpu.sync_copy(exb.at[p, j],
                                        den_sh.at[dstb.at[j]], add=True)
                    return carry

                lax.fori_loop(0, NCH, jbody2, 0)
                plsc.subcore_barrier()

                hc = h * csplit + half
                pltpu.sync_copy(acc_sh.at[pl.ds(row0, STRIPE)],
                                acc_out.at[hc, pl.ds(row0, STRIPE)])
                if half == 0:
                    pltpu.sync_copy(
                        den_sh.at[pl.ds(row0, STRIPE)],
                        den_out.at[pl.ds(h * NPAD + row0, STRIPE)])
                plsc.subcore_barrier()

        if with_alpha:
            def phase3(den_v, alphab):
                pltpu.sync_copy(den_sh, den_v)

                def jb(j, carry):
                    for v in range(CHUNK // 16):
                        dv = dstb[j, pl.ds(v * 16, 16)]
                        ex = exb[0, j, pl.ds(v * 16, 16)]
                        dnv = plsc.load_gather(den_v, [dv])
                        alphab[j, pl.ds(v * 16, 16)] = ex / (dnv + 1e-16)
                    return carry

                lax.fori_loop(0, NCH, jb, 0)
                pltpu.sync_copy(alphab, alpha_out.at[sid])

            pl.run_scoped(phase3,
                          pltpu.VMEM((NPAD,), jnp.float32),
                          pltpu.VMEM((NCH, CHUNK), jnp.float32))

    if hpc * NC > heads_total:
        @pl.when(cid == 0)
        def _():
            tile_work()
    else:
        tile_work()


def _edge_call(src2d, dst2d, ssrc, sdst, xpflat, zrows, zden,
               heads_total, hpc, c, csplit, with_alpha):
    cfg = (hpc, heads_total, c, csplit, with_alpha)
    out_type = [
        jax.ShapeDtypeStruct((heads_total * csplit, NPAD, c), jnp.float32),
        jax.ShapeDtypeStruct((heads_total * NPAD,), jnp.float32),
    ]
    if with_alpha:
        out_type.append(jax.ShapeDtypeStruct((NS, NCH, CHUNK),
                                             jnp.float32))
    scratch = [
        pltpu.VMEM((NCH, CHUNK), jnp.int32),
        pltpu.VMEM((NCH, CHUNK), jnp.int32),
        pltpu.VMEM((hpc, NCH, CHUNK), jnp.float32),
        pltpu.VMEM((CHUNK,), jnp.int32),
        pltpu.VMEM((CHUNK, c), jnp.float32),
        pltpu.SemaphoreType.DMA,
        pltpu.VMEM_SHARED((NPAD, c), jnp.float32),
        pltpu.VMEM_SHARED((NPAD,), jnp.float32),
    ]
    mesh = plsc.VectorSubcoreMesh(core_axis_name="c", subcore_axis_name="s",
                                  num_cores=NC, num_subcores=NS)
    fn = pl.kernel(functools.partial(_edge_body, cfg),
                   out_type=out_type, mesh=mesh, scratch_types=scratch,
                   compiler_params=pltpu.CompilerParams(
                       needs_layout_passes=False,
                       use_tc_tiling_on_sc=False))
    return fn(src2d, dst2d, ssrc, sdst, xpflat, zrows, zden)


def kernel(x, edge_index, W1, a_src1, a_dst1, b1, ln1_g, ln1_b,
           W2, a_src2, a_dst2, b2, ln2_g, ln2_b,
           W3, a_src3, a_dst3, b3, lno_g, lno_b):
    f32 = jnp.float32
    src2d = edge_index[0].reshape(NS, NCH, CHUNK)
    dst2d = edge_index[1].reshape(NS, NCH, CHUNK)

    def attn_mat(a_s, a_d, nheads, c, cp):
        a = jnp.zeros((nheads * cp, 2 * nheads), f32)
        for hh in range(nheads):
            a = a.at[hh * cp:hh * cp + c, hh].set(a_s[hh])
            a = a.at[hh * cp:hh * cp + c, nheads + hh].set(a_d[hh])
        return a

    A1 = attn_mat(a_src1, a_dst1, H, C1, C1)
    A2 = attn_mat(a_src2, a_dst2, H, C2, C2P)
    A3 = attn_mat(a_src3, a_dst3, 1, C3, C3P)

    W2p = jnp.zeros((G1, G2P), f32)
    for hh in range(H):
        W2p = W2p.at[:, hh * C2P:hh * C2P + C2].set(
            W2[:, hh * C2:hh * C2 + C2])
    W3p = jnp.zeros((G2P, C3P), f32)
    for hh in range(H):
        W3p = W3p.at[hh * C2P:hh * C2P + C2, :C3].set(
            W3[hh * C2:hh * C2 + C2, :])

    def padvec(v, nheads, c, cp):
        o = jnp.zeros((1, nheads * cp), f32)
        for hh in range(nheads):
            o = o.at[0, hh * cp:hh * cp + c].set(v[hh * c:hh * c + c])
        return o

    b1r = b1.reshape(1, G1)
    g1r = ln1_g.reshape(1, G1)
    l1r = ln1_b.reshape(1, G1)
    m1 = jnp.ones((1, G1), f32)
    b2r = padvec(b2, H, C2, C2P)
    g2r = padvec(ln2_g, H, C2, C2P)
    l2r = padvec(ln2_b, H, C2, C2P)
    m2 = padvec(jnp.ones((H * C2,), f32), H, C2, C2P)
    b3r = padvec(b3, 1, C3, C3P)
    g3r = padvec(lno_g, 1, C3, C3P)
    l3r = padvec(lno_b, 1, C3, C3P)
    m3 = padvec(jnp.ones((C3,), f32), 1, C3, C3P)

    zr1 = jnp.zeros((STRIPE, C1 // 2), f32)
    zr2 = jnp.zeros((STRIPE, C2P), f32)
    zden = jnp.zeros((STRIPE,), f32)

    xp1, s1 = _mm1(x, W1, A1)
    acc1, den1 = _edge_call(src2d, dst2d, s1[:, :H].T, s1[:, H:].T,
                            xp1.reshape(N * H * 2, C1 // 2), zr1, zden,
                            H, 2, C1 // 2, 2, False)

    xp2, s2 = _post_mm(2 * H, 2, G1, C1 // 2, G2P, 2 * H,
                       acc1[:, :N, :], den1.reshape(H, NPAD)[:, :N].T,
                       b1r, g1r, l1r, m1, W2p, A2)
    acc2, den2 = _edge_call(src2d, dst2d, s2[:, :H].T, s2[:, H:].T,
                            xp2.reshape(N * H, C2P), zr2, zden,
                            H, 2, C2P, 1, False)

    xp3, s3 = _post_mm(H, 1, H * C2, C2P, C3P, 2,
                       acc2[:, :N, :], den2.reshape(H, NPAD)[:, :N].T,
                       b2r, g2r, l2r, m2, W3p, A3)
    acc3, den3, alpha = _edge_call(src2d, dst2d, s3[:, 0], s3[:, 1],
                                   xp3, zr2, zden, 1, 1, C3P, 1, True)

    out = _post3(acc3[:, :N, :], den3.reshape(1, NPAD)[:, :N].T,
                 b3r, g3r, l3r, m3)

    h1 = xp1.reshape(N, H, C1)
    h2 = xp2.reshape(N, H, C2P)[:, :, :C2]
    h3 = xp3[:, :C3].reshape(N, 1, C3)
    return (out, h1, h2, h3, alpha.reshape(E, 1))


# speedup vs baseline: 26.2061x; 1.3869x over previous
import functools

import jax
import jax.numpy as jnp
from jax import lax
from jax.experimental import pallas as pl
from jax.experimental.pallas import tpu as pltpu
from jax.experimental.pallas import tpu_sc as plsc

N = 10000
E = 160000
F = 256
H = 4
C1 = 128
C2 = 40
C2P = 48
C3 = 40
C3P = 48
G1 = H * C1
G2P = H * C2P
NC = 2
NS = 16
CHUNK = 80
NCH = (E // NS) // CHUNK
NPAD = 10240
STRIPE = NPAD // NS
LAST_STRIPE = N - (NS - 1) * STRIPE

_HIGH = jax.lax.Precision.HIGHEST


def _dot(a, b):
    return jnp.dot(a, b, preferred_element_type=jnp.float32, precision=_HIGH)


def _mm1_body(x_ref, w_ref, a_ref, xp_ref, s_ref):
    xp = _dot(x_ref[:, :], w_ref[:, :])
    xp_ref[:, :] = xp
    s_ref[:, :] = _dot(xp, a_ref[:, :])


def _post_mm_body(nblocks, csplit, nreal, acc_ref, den_ref, b_ref, g_ref,
                  bl_ref, msk_ref, w_ref, a_ref, xp_ref, s_ref):
    den = den_ref[:, :]
    ys = []
    for hc in range(nblocks):
        h = hc // csplit
        d = den[:, h:h + 1] + 1e-16
        ys.append(acc_ref[hc, :, :] / d)
    y = jnp.concatenate(ys, axis=-1) + b_ref[:, :]
    inv = 1.0 / float(nreal)
    mu = jnp.sum(y, axis=-1, keepdims=True) * inv
    dlt = y - mu
    var = jnp.sum(msk_ref[:, :] * dlt * dlt, axis=-1, keepdims=True) * inv
    z = dlt * jax.lax.rsqrt(var + 1e-5) * g_ref[:, :] + bl_ref[:, :]
    act = jnp.where(z > 0, z, jnp.exp(z) - 1.0)
    xp = _dot(act, w_ref[:, :])
    xp_ref[:, :] = xp
    s_ref[:, :] = _dot(xp, a_ref[:, :])


def _post3_body(acc_ref, den_ref, b_ref, g_ref, bl_ref, msk_ref, out_ref):
    d = den_ref[:, :] + 1e-16
    y = acc_ref[0, :, :] / d + b_ref[:, :]
    inv = 1.0 / float(C3)
    mu = jnp.sum(y, axis=-1, keepdims=True) * inv
    dlt = y - mu
    var = jnp.sum(msk_ref[:, :] * dlt * dlt, axis=-1, keepdims=True) * inv
    z = dlt * jax.lax.rsqrt(var + 1e-5) * g_ref[:, :] + bl_ref[:, :]
    out_ref[:, :] = z[:, :C3]


_BN = 2000
_GRID = N // _BN


def _row_spec(cols):
    return pl.BlockSpec((_BN, cols), lambda i: (i, 0))


def _full_spec(r, c):
    return pl.BlockSpec((r, c), lambda i: (0, 0))


def _mm1(x, w, a):
    return pl.pallas_call(
        _mm1_body,
        grid=(_GRID,),
        in_specs=[_row_spec(F), _full_spec(F, G1), _full_spec(G1, 2 * H)],
        out_specs=[_row_spec(G1), _row_spec(2 * H)],
        out_shape=[jax.ShapeDtypeStruct((N, G1), jnp.float32),
                   jax.ShapeDtypeStruct((N, 2 * H), jnp.float32)],
    )(x, w, a)


def _post_mm(nblocks, csplit, nreal, cin, gout, nsl, acc, den, b, g, bl,
             msk, w, a):
    body = functools.partial(_post_mm_body, nblocks, csplit, nreal)
    gin = nblocks * cin
    nh = nblocks // csplit
    return pl.pallas_call(
        body,
        grid=(_GRID,),
        in_specs=[
            pl.BlockSpec((nblocks, _BN, cin), lambda i: (0, i, 0)),
            pl.BlockSpec((_BN, nh), lambda i: (i, 0)),
            _full_spec(1, gin), _full_spec(1, gin), _full_spec(1, gin),
            _full_spec(1, gin), _full_spec(gin, gout), _full_spec(gout, nsl),
        ],
        out_specs=[_row_spec(gout), _row_spec(nsl)],
        out_shape=[jax.ShapeDtypeStruct((N, gout), jnp.float32),
                   jax.ShapeDtypeStruct((N, nsl), jnp.float32)],
    )(acc, den, b, g, bl, msk, w, a)


def _post3(acc, den, b, g, bl, msk):
    return pl.pallas_call(
        _post3_body,
        grid=(_GRID,),
        in_specs=[
            pl.BlockSpec((1, _BN, C3P), lambda i: (0, i, 0)),
            pl.BlockSpec((_BN, 1), lambda i: (i, 0)),
            _full_spec(1, C3P), _full_spec(1, C3P), _full_spec(1, C3P),
            _full_spec(1, C3P),
        ],
        out_specs=_row_spec(C3),
        out_shape=jax.ShapeDtypeStruct((N, C3), jnp.float32),
    )(acc, den, b, g, bl, msk)


def _edge_body(cfg, src_hbm, dst_hbm, ssrc_hbm, xp_hbm,
               zrows_hbm, zden_hbm, *rest):
    hpc, heads_total, c, csplit, with_alpha = cfg
    if with_alpha:
        acc_out, den_out, alpha_out, srcb, dstb, exb, sem, acc_sh, \
            den_sh = rest
    else:
        acc_out, den_out, srcb, dstb, exb, sem, acc_sh, den_sh = rest

    cid = lax.axis_index("c")
    sid = lax.axis_index("s")

    def tile_work():
        pltpu.sync_copy(src_hbm.at[sid], srcb)
        pltpu.sync_copy(dst_hbm.at[sid], dstb)

        row0 = sid * STRIPE

        def phase2(rows2, gidx2):
            for p in range(hpc):
                h = cid * hpc + p

                def phase1(tsrc_v, tdst_v):
                    pltpu.sync_copy(ssrc_hbm.at[h], tsrc_v)
                    pltpu.sync_copy(ssrc_hbm.at[h + heads_total], tdst_v)

                    def jbody(j, carry):
                        for v in range(CHUNK // 16):
                            sv = srcb[j, pl.ds(v * 16, 16)]
                            dv = dstb[j, pl.ds(v * 16, 16)]
                            e = (plsc.load_gather(tsrc_v, [sv])
                                 + plsc.load_gather(tdst_v, [dv]))
                            e = jnp.where(e >= 0, e, e * jnp.float32(0.2))
                            exb[j, pl.ds(v * 16, 16)] = jnp.exp(e)
                        return carry

                    lax.fori_loop(0, NCH, jbody, 0)

                pl.run_scoped(phase1,
                              pltpu.VMEM((N,), jnp.float32),
                              pltpu.VMEM((N,), jnp.float32))

                for half in range(csplit):
                    pltpu.sync_copy(zrows_hbm,
                                    acc_sh.at[pl.ds(row0, STRIPE)])
                    if half == 0:
                        pltpu.sync_copy(zden_hbm,
                                        den_sh.at[pl.ds(row0, STRIPE)])
                    plsc.subcore_barrier()

                    def start_gather(j, par):
                        for v in range(CHUNK // 16):
                            sv = srcb[j, pl.ds(v * 16, 16)]
                            gidx2[par, pl.ds(v * 16, 16)] = (
                                (sv * heads_total + h) * csplit + half)
                        pltpu.async_copy(xp_hbm.at[gidx2.at[par]],
                                         rows2.at[par], sem.at[par])

                    start_gather(0, 0)

                    def jbody2(j, carry):
                        par = lax.rem(j, 2)

                        @pl.when(j + 1 < NCH)
                        def _():
                            start_gather(j + 1, 1 - par)

                        pltpu.make_async_copy(
                            xp_hbm.at[gidx2.at[par]], rows2.at[par],
                            sem.at[par]).wait()

                        def rbody(v, rc):
                            ex16 = exb[j, pl.ds(v * 16, 16)]
                            for r16 in range(16):
                                r = v * 16 + r16
                                s = ex16[r16]
                                for k in range(c // 16):
                                    rows2[par, r, pl.ds(k * 16, 16)] = (
                                        rows2[par, r, pl.ds(k * 16, 16)]
                                        * s)
                            return rc

                        lax.fori_loop(0, CHUNK // 16, rbody, 0)
                        pltpu.sync_copy(rows2.at[par],
                                        acc_sh.at[dstb.at[j]], add=True)
                        if half == 0:
                            pltpu.sync_copy(exb.at[j],
                                            den_sh.at[dstb.at[j]],
                                            add=True)
                        return carry

                    lax.fori_loop(0, NCH, jbody2, 0)
                    plsc.subcore_barrier()

                    hc = h * csplit + half
                    pltpu.sync_copy(acc_sh.at[pl.ds(row0, STRIPE)],
                                    acc_out.at[hc, pl.ds(row0, STRIPE)])
                    if half == 0:
                        pltpu.sync_copy(
                            den_sh.at[pl.ds(row0, STRIPE)],
                            den_out.at[pl.ds(h * NPAD + row0, STRIPE)])
                    plsc.subcore_barrier()

        pl.run_scoped(phase2,
                      pltpu.VMEM((2, CHUNK, c), jnp.float32),
                      pltpu.VMEM((2, CHUNK), jnp.int32))

        if with_alpha:
            def phase3(den_v, alphab):
                pltpu.sync_copy(den_sh, den_v)

                def jb(j, carry):
                    for v in range(CHUNK // 16):
                        dv = dstb[j, pl.ds(v * 16, 16)]
                        ex = exb[j, pl.ds(v * 16, 16)]
                        dnv = plsc.load_gather(den_v, [dv])
                        alphab[j, pl.ds(v * 16, 16)] = ex / (dnv + 1e-16)
                    return carry

                lax.fori_loop(0, NCH, jb, 0)
                pltpu.sync_copy(alphab, alpha_out.at[sid])

            pl.run_scoped(phase3,
                          pltpu.VMEM((NPAD,), jnp.float32),
                          pltpu.VMEM((NCH, CHUNK), jnp.float32))

    if hpc * NC > heads_total:
        @pl.when(cid == 0)
        def _():
            tile_work()
    else:
        tile_work()


def _edge_call(src2d, dst2d, stab, xpflat, zrows, zden,
               heads_total, hpc, c, csplit, with_alpha):
    cfg = (hpc, heads_total, c, csplit, with_alpha)
    out_type = [
        jax.ShapeDtypeStruct((heads_total * csplit, NPAD, c), jnp.float32),
        jax.ShapeDtypeStruct((heads_total * NPAD,), jnp.float32),
    ]
    if with_alpha:
        out_type.append(jax.ShapeDtypeStruct((NS, NCH, CHUNK),
                                             jnp.float32))
    scratch = [
        pltpu.VMEM((NCH, CHUNK), jnp.int32),
        pltpu.VMEM((NCH, CHUNK), jnp.int32),
        pltpu.VMEM((NCH, CHUNK), jnp.float32),
        pltpu.SemaphoreType.DMA((2,)),
        pltpu.VMEM_SHARED((NPAD, c), jnp.float32),
        pltpu.VMEM_SHARED((NPAD,), jnp.float32),
    ]
    mesh = plsc.VectorSubcoreMesh(core_axis_name="c", subcore_axis_name="s",
                                  num_cores=NC, num_subcores=NS)
    fn = pl.kernel(functools.partial(_edge_body, cfg),
                   out_type=out_type, mesh=mesh, scratch_types=scratch,
                   compiler_params=pltpu.CompilerParams(
                       needs_layout_passes=False,
                       use_tc_tiling_on_sc=False))
    return fn(src2d, dst2d, stab, xpflat, zrows, zden)


def kernel(x, edge_index, W1, a_src1, a_dst1, b1, ln1_g, ln1_b,
           W2, a_src2, a_dst2, b2, ln2_g, ln2_b,
           W3, a_src3, a_dst3, b3, lno_g, lno_b):
    f32 = jnp.float32
    src2d = edge_index[0].reshape(NS, NCH, CHUNK)
    dst2d = edge_index[1].reshape(NS, NCH, CHUNK)

    def attn_mat(a_s, a_d, nheads, c, cp):
        a = jnp.zeros((nheads * cp, 2 * nheads), f32)
        for hh in range(nheads):
            a = a.at[hh * cp:hh * cp + c, hh].set(a_s[hh])
            a = a.at[hh * cp:hh * cp + c, nheads + hh].set(a_d[hh])
        return a

    A1 = attn_mat(a_src1, a_dst1, H, C1, C1)
    A2 = attn_mat(a_src2, a_dst2, H, C2, C2P)
    A3 = attn_mat(a_src3, a_dst3, 1, C3, C3P)

    W2p = jnp.zeros((G1, G2P), f32)
    for hh in range(H):
        W2p = W2p.at[:, hh * C2P:hh * C2P + C2].set(
            W2[:, hh * C2:hh * C2 + C2])
    W3p = jnp.zeros((G2P, C3P), f32)
    for hh in range(H):
        W3p = W3p.at[hh * C2P:hh * C2P + C2, :C3].set(
            W3[hh * C2:hh * C2 + C2, :])

    def padvec(v, nheads, c, cp):
        o = jnp.zeros((1, nheads * cp), f32)
        for hh in range(nheads):
            o = o.at[0, hh * cp:hh * cp + c].set(v[hh * c:hh * c + c])
        return o

    b1r = b1.reshape(1, G1)
    g1r = ln1_g.reshape(1, G1)
    l1r = ln1_b.reshape(1, G1)
    m1 = jnp.ones((1, G1), f32)
    b2r = padvec(b2, H, C2, C2P)
    g2r = padvec(ln2_g, H, C2, C2P)
    l2r = padvec(ln2_b, H, C2, C2P)
    m2 = padvec(jnp.ones((H * C2,), f32), H, C2, C2P)
    b3r = padvec(b3, 1, C3, C3P)
    g3r = padvec(lno_g, 1, C3, C3P)
    l3r = padvec(lno_b, 1, C3, C3P)
    m3 = padvec(jnp.ones((C3,), f32), 1, C3, C3P)

    zr1 = jnp.zeros((STRIPE, C1 // 2), f32)
    zr2 = jnp.zeros((STRIPE, C2P), f32)
    zden = jnp.zeros((STRIPE,), f32)

    xp1, s1 = _mm1(x, W1, A1)
    acc1, den1 = _edge_call(src2d, dst2d, s1.T,
                            xp1.reshape(N * H * 2, C1 // 2), zr1, zden,
                            H, 2, C1 // 2, 2, False)

    xp2, s2 = _post_mm(2 * H, 2, G1, C1 // 2, G2P, 2 * H,
                       acc1[:, :N, :], den1.reshape(H, NPAD)[:, :N].T,
                       b1r, g1r, l1r, m1, W2p, A2)
    acc2, den2 = _edge_call(src2d, dst2d, s2.T,
                            xp2.reshape(N * H, C2P), zr2, zden,
                            H, 2, C2P, 1, False)

    xp3, s3 = _post_mm(H, 1, H * C2, C2P, C3P, 2,
                       acc2[:, :N, :], den2.reshape(H, NPAD)[:, :N].T,
                       b2r, g2r, l2r, m2, W3p, A3)
    acc3, den3, alpha = _edge_call(src2d, dst2d, s3.T,
                                   xp3, zr2, zden, 1, 1, C3P, 1, True)

    out = _post3(acc3[:, :N, :], den3.reshape(1, NPAD)[:, :N].T,
                 b3r, g3r, l3r, m3)

    h1 = xp1.reshape(N, H, C1)
    h2 = xp2.reshape(N, H, C2P)[:, :, :C2]
    h3 = xp3[:, :C3].reshape(N, 1, C3)
    return (out, h1, h2, h3, alpha.reshape(E, 1))


# speedup vs baseline: 27.9138x; 1.0652x over previous
import functools

import jax
import jax.numpy as jnp
from jax import lax
from jax.experimental import pallas as pl
from jax.experimental.pallas import tpu as pltpu
from jax.experimental.pallas import tpu_sc as plsc

N = 10000
E = 160000
F = 256
H = 4
C1 = 128
C2 = 40
C2P = 48
C3 = 40
C3P = 48
G1 = H * C1
G2P = H * C2P
NC = 2
NS = 16
CHUNK = 80
NCH = (E // NS) // CHUNK
NB = 4
NPAD = 10240
STRIPE = NPAD // NS
LAST_STRIPE = N - (NS - 1) * STRIPE

_HIGH = jax.lax.Precision.HIGHEST


def _dot(a, b):
    return jnp.dot(a, b, preferred_element_type=jnp.float32, precision=_HIGH)


def _mm1_body(x_ref, w_ref, a_ref, xp_ref, s_ref):
    xp = _dot(x_ref[:, :], w_ref[:, :])
    xp_ref[:, :] = xp
    s_ref[:, :] = _dot(xp, a_ref[:, :])


def _post_mm_body(nblocks, csplit, nreal, acc_ref, den_ref, b_ref, g_ref,
                  bl_ref, msk_ref, w_ref, a_ref, xp_ref, s_ref):
    den = den_ref[:, :]
    ys = []
    for hc in range(nblocks):
        h = hc // csplit
        d = den[:, h:h + 1] + 1e-16
        ys.append(acc_ref[hc, :, :] / d)
    y = jnp.concatenate(ys, axis=-1) + b_ref[:, :]
    inv = 1.0 / float(nreal)
    mu = jnp.sum(y, axis=-1, keepdims=True) * inv
    dlt = y - mu
    var = jnp.sum(msk_ref[:, :] * dlt * dlt, axis=-1, keepdims=True) * inv
    z = dlt * jax.lax.rsqrt(var + 1e-5) * g_ref[:, :] + bl_ref[:, :]
    act = jnp.where(z > 0, z, jnp.exp(z) - 1.0)
    xp = _dot(act, w_ref[:, :])
    xp_ref[:, :] = xp
    s_ref[:, :] = _dot(xp, a_ref[:, :])


def _post3_body(acc_ref, den_ref, b_ref, g_ref, bl_ref, msk_ref, out_ref):
    d = den_ref[:, :] + 1e-16
    y = acc_ref[0, :, :] / d + b_ref[:, :]
    inv = 1.0 / float(C3)
    mu = jnp.sum(y, axis=-1, keepdims=True) * inv
    dlt = y - mu
    var = jnp.sum(msk_ref[:, :] * dlt * dlt, axis=-1, keepdims=True) * inv
    z = dlt * jax.lax.rsqrt(var + 1e-5) * g_ref[:, :] + bl_ref[:, :]
    out_ref[:, :] = z[:, :C3]


_BN = 2000
_GRID = N // _BN


def _row_spec(cols):
    return pl.BlockSpec((_BN, cols), lambda i: (i, 0))


def _full_spec(r, c):
    return pl.BlockSpec((r, c), lambda i: (0, 0))


def _mm1(x, w, a):
    return pl.pallas_call(
        _mm1_body,
        grid=(_GRID,),
        in_specs=[_row_spec(F), _full_spec(F, G1), _full_spec(G1, 2 * H)],
        out_specs=[_row_spec(G1), _row_spec(2 * H)],
        out_shape=[jax.ShapeDtypeStruct((N, G1), jnp.float32),
                   jax.ShapeDtypeStruct((N, 2 * H), jnp.float32)],
    )(x, w, a)


def _post_mm(nblocks, csplit, nreal, cin, gout, nsl, acc, den, b, g, bl,
             msk, w, a):
    body = functools.partial(_post_mm_body, nblocks, csplit, nreal)
    gin = nblocks * cin
    nh = nblocks // csplit
    return pl.pallas_call(
        body,
        grid=(_GRID,),
        in_specs=[
            pl.BlockSpec((nblocks, _BN, cin), lambda i: (0, i, 0)),
            pl.BlockSpec((_BN, nh), lambda i: (i, 0)),
            _full_spec(1, gin), _full_spec(1, gin), _full_spec(1, gin),
            _full_spec(1, gin), _full_spec(gin, gout), _full_spec(gout, nsl),
        ],
        out_specs=[_row_spec(gout), _row_spec(nsl)],
        out_shape=[jax.ShapeDtypeStruct((N, gout), jnp.float32),
                   jax.ShapeDtypeStruct((N, nsl), jnp.float32)],
    )(acc, den, b, g, bl, msk, w, a)


def _post3(acc, den, b, g, bl, msk):
    return pl.pallas_call(
        _post3_body,
        grid=(_GRID,),
        in_specs=[
            pl.BlockSpec((1, _BN, C3P), lambda i: (0, i, 0)),
            pl.BlockSpec((_BN, 1), lambda i: (i, 0)),
            _full_spec(1, C3P), _full_spec(1, C3P), _full_spec(1, C3P),
            _full_spec(1, C3P),
        ],
        out_specs=_row_spec(C3),
        out_shape=jax.ShapeDtypeStruct((N, C3), jnp.float32),
    )(acc, den, b, g, bl, msk)


def _edge_body(cfg, src_hbm, dst_hbm, ssrc_hbm, xp_hbm,
               zrows_hbm, zden_hbm, *rest):
    hpc, heads_total, c, csplit, with_alpha = cfg
    if with_alpha:
        acc_out, den_out, alpha_out, srcb, dstb, exb, gsem, ssem, \
            acc_sh, den_sh = rest
    else:
        acc_out, den_out, srcb, dstb, exb, gsem, ssem, acc_sh, \
            den_sh = rest

    cid = lax.axis_index("c")
    sid = lax.axis_index("s")

    def tile_work():
        pltpu.sync_copy(src_hbm.at[sid], srcb)
        pltpu.sync_copy(dst_hbm.at[sid], dstb)

        row0 = sid * STRIPE

        def phase2(rows4, gidx4):
            for p in range(hpc):
                h = cid * hpc + p

                def phase1(tsrc_v, tdst_v):
                    pltpu.sync_copy(ssrc_hbm.at[h], tsrc_v)
                    pltpu.sync_copy(ssrc_hbm.at[h + heads_total], tdst_v)

                    def jbody(j, carry):
                        for v in range(CHUNK // 16):
                            sv = srcb[j, pl.ds(v * 16, 16)]
                            dv = dstb[j, pl.ds(v * 16, 16)]
                            e = (plsc.load_gather(tsrc_v, [sv])
                                 + plsc.load_gather(tdst_v, [dv]))
                            e = jnp.where(e >= 0, e, e * jnp.float32(0.2))
                            exb[j, pl.ds(v * 16, 16)] = jnp.exp(e)
                        return carry

                    lax.fori_loop(0, NCH, jbody, 0)

                pl.run_scoped(phase1,
                              pltpu.VMEM((N,), jnp.float32),
                              pltpu.VMEM((N,), jnp.float32))

                for half in range(csplit):
                    pltpu.sync_copy(zrows_hbm,
                                    acc_sh.at[pl.ds(row0, STRIPE)])
                    if half == 0:
                        pltpu.sync_copy(zden_hbm,
                                        den_sh.at[pl.ds(row0, STRIPE)])
                    plsc.subcore_barrier()

                    def mk_gather(b):
                        return pltpu.make_async_copy(
                            xp_hbm.at[gidx4.at[b]], rows4.at[b],
                            gsem.at[b])

                    def mk_scatter(b, g):
                        return pltpu.make_async_copy(
                            rows4.at[b], acc_sh.at[dstb.at[g]],
                            ssem.at[b])

                    def start_gather(g):
                        b = lax.rem(g, NB)
                        for v in range(CHUNK // 16):
                            sv = srcb[g, pl.ds(v * 16, 16)]
                            gidx4[b, pl.ds(v * 16, 16)] = (
                                (sv * heads_total + h) * csplit + half)
                        mk_gather(b).start()

                    for g0 in range(NB - 1):
                        start_gather(jnp.int32(g0))

                    def jbody2(j, carry):
                        b = lax.rem(j, NB)

                        @pl.when(j + (NB - 1) < NCH)
                        def _():
                            @pl.when(j >= 1)
                            def _():
                                mk_scatter(lax.rem(j - 1, NB),
                                           j - 1).wait()
                            start_gather(j + (NB - 1))

                        mk_gather(b).wait()

                        def rbody(v, rc):
                            ex16 = exb[j, pl.ds(v * 16, 16)]
                            for r16 in range(16):
                                r = v * 16 + r16
                                s = ex16[r16]
                                for k in range(c // 16):
                                    rows4[b, r, pl.ds(k * 16, 16)] = (
                                        rows4[b, r, pl.ds(k * 16, 16)]
                                        * s)
                            return rc

                        lax.fori_loop(0, CHUNK // 16, rbody, 0)
                        mk_scatter(b, j).start(add=True)
                        if half == 0:
                            pltpu.sync_copy(exb.at[j],
                                            den_sh.at[dstb.at[j]],
                                            add=True)
                        return carry

                    lax.fori_loop(0, NCH, jbody2, 0)
                    for t in range(NB):
                        g = jnp.int32(NCH - NB + t)
                        mk_scatter(lax.rem(g, NB), g).wait()
                    plsc.subcore_barrier()

                    hc = h * csplit + half
                    pltpu.sync_copy(acc_sh.at[pl.ds(row0, STRIPE)],
                                    acc_out.at[hc, pl.ds(row0, STRIPE)])
                    if half == 0:
                        pltpu.sync_copy(
                            den_sh.at[pl.ds(row0, STRIPE)],
                            den_out.at[pl.ds(h * NPAD + row0, STRIPE)])
                    plsc.subcore_barrier()

        pl.run_scoped(phase2,
                      pltpu.VMEM((NB, CHUNK, c), jnp.float32),
                      pltpu.VMEM((NB, CHUNK), jnp.int32))

        if with_alpha:
            def phase3(den_v, alphab):
                pltpu.sync_copy(den_sh, den_v)

                def jb(j, carry):
                    for v in range(CHUNK // 16):
                        dv = dstb[j, pl.ds(v * 16, 16)]
                        ex = exb[j, pl.ds(v * 16, 16)]
                        dnv = plsc.load_gather(den_v, [dv])
                        alphab[j, pl.ds(v * 16, 16)] = ex / (dnv + 1e-16)
                    return carry

                lax.fori_loop(0, NCH, jb, 0)
                pltpu.sync_copy(alphab, alpha_out.at[sid])

            pl.run_scoped(phase3,
                          pltpu.VMEM((NPAD,), jnp.float32),
                          pltpu.VMEM((NCH, CHUNK), jnp.float32))

    if hpc * NC > heads_total:
        @pl.when(cid == 0)
        def _():
            tile_work()
    else:
        tile_work()


def _edge_call(src2d, dst2d, stab, xpflat, zrows, zden,
               heads_total, hpc, c, csplit, with_alpha):
    cfg = (hpc, heads_total, c, csplit, with_alpha)
    out_type = [
        jax.ShapeDtypeStruct((heads_total * csplit, NPAD, c), jnp.float32),
        jax.ShapeDtypeStruct((heads_total * NPAD,), jnp.float32),
    ]
    if with_alpha:
        out_type.append(jax.ShapeDtypeStruct((NS, NCH, CHUNK),
                                             jnp.float32))
    scratch = [
        pltpu.VMEM((NCH, CHUNK), jnp.int32),
        pltpu.VMEM((NCH, CHUNK), jnp.int32),
        pltpu.VMEM((NCH, CHUNK), jnp.float32),
        pltpu.SemaphoreType.DMA((NB,)),
        pltpu.SemaphoreType.DMA((NB,)),
        pltpu.VMEM_SHARED((NPAD, c), jnp.float32),
        pltpu.VMEM_SHARED((NPAD,), jnp.float32),
    ]
    mesh = plsc.VectorSubcoreMesh(core_axis_name="c", subcore_axis_name="s",
                                  num_cores=NC, num_subcores=NS)
    fn = pl.kernel(functools.partial(_edge_body, cfg),
                   out_type=out_type, mesh=mesh, scratch_types=scratch,
                   compiler_params=pltpu.CompilerParams(
                       needs_layout_passes=False,
                       use_tc_tiling_on_sc=False))
    return fn(src2d, dst2d, stab, xpflat, zrows, zden)


def kernel(x, edge_index, W1, a_src1, a_dst1, b1, ln1_g, ln1_b,
           W2, a_src2, a_dst2, b2, ln2_g, ln2_b,
           W3, a_src3, a_dst3, b3, lno_g, lno_b):
    f32 = jnp.float32
    src2d = edge_index[0].reshape(NS, NCH, CHUNK)
    dst2d = edge_index[1].reshape(NS, NCH, CHUNK)

    def attn_mat(a_s, a_d, nheads, c, cp):
        a = jnp.zeros((nheads * cp, 2 * nheads), f32)
        for hh in range(nheads):
            a = a.at[hh * cp:hh * cp + c, hh].set(a_s[hh])
            a = a.at[hh * cp:hh * cp + c, nheads + hh].set(a_d[hh])
        return a

    A1 = attn_mat(a_src1, a_dst1, H, C1, C1)
    A2 = attn_mat(a_src2, a_dst2, H, C2, C2P)
    A3 = attn_mat(a_src3, a_dst3, 1, C3, C3P)

    W2p = jnp.zeros((G1, G2P), f32)
    for hh in range(H):
        W2p = W2p.at[:, hh * C2P:hh * C2P + C2].set(
            W2[:, hh * C2:hh * C2 + C2])
    W3p = jnp.zeros((G2P, C3P), f32)
    for hh in range(H):
        W3p = W3p.at[hh * C2P:hh * C2P + C2, :C3].set(
            W3[hh * C2:hh * C2 + C2, :])

    def padvec(v, nheads, c, cp):
        o = jnp.zeros((1, nheads * cp), f32)
        for hh in range(nheads):
            o = o.at[0, hh * cp:hh * cp + c].set(v[hh * c:hh * c + c])
        return o

    b1r = b1.reshape(1, G1)
    g1r = ln1_g.reshape(1, G1)
    l1r = ln1_b.reshape(1, G1)
    m1 = jnp.ones((1, G1), f32)
    b2r = padvec(b2, H, C2, C2P)
    g2r = padvec(ln2_g, H, C2, C2P)
    l2r = padvec(ln2_b, H, C2, C2P)
    m2 = padvec(jnp.ones((H * C2,), f32), H, C2, C2P)
    b3r = padvec(b3, 1, C3, C3P)
    g3r = padvec(lno_g, 1, C3, C3P)
    l3r = padvec(lno_b, 1, C3, C3P)
    m3 = padvec(jnp.ones((C3,), f32), 1, C3, C3P)

    zr1 = jnp.zeros((STRIPE, C1 // 2), f32)
    zr2 = jnp.zeros((STRIPE, C2P), f32)
    zden = jnp.zeros((STRIPE,), f32)

    xp1, s1 = _mm1(x, W1, A1)
    acc1, den1 = _edge_call(src2d, dst2d, s1.T,
                            xp1.reshape(N * H * 2, C1 // 2), zr1, zden,
                            H, 2, C1 // 2, 2, False)

    xp2, s2 = _post_mm(2 * H, 2, G1, C1 // 2, G2P, 2 * H,
                       acc1[:, :N, :], den1.reshape(H, NPAD)[:, :N].T,
                       b1r, g1r, l1r, m1, W2p, A2)
    acc2, den2 = _edge_call(src2d, dst2d, s2.T,
                            xp2.reshape(N * H, C2P), zr2, zden,
                            H, 2, C2P, 1, False)

    xp3, s3 = _post_mm(H, 1, H * C2, C2P, C3P, 2,
                       acc2[:, :N, :], den2.reshape(H, NPAD)[:, :N].T,
                       b2r, g2r, l2r, m2, W3p, A3)
    acc3, den3, alpha = _edge_call(src2d, dst2d, s3.T,
                                   xp3, zr2, zden, 1, 1, C3P, 1, True)

    out = _post3(acc3[:, :N, :], den3.reshape(1, NPAD)[:, :N].T,
                 b3r, g3r, l3r, m3)

    h1 = xp1.reshape(N, H, C1)
    h2 = xp2.reshape(N, H, C2P)[:, :, :C2]
    h3 = xp3[:, :C3].reshape(N, 1, C3)
    return (out, h1, h2, h3, alpha.reshape(E, 1))
